# baseline (speedup 1.0000x reference)
import functools

import jax
import jax.numpy as jnp
from jax import lax
from jax.experimental import pallas as pl
from jax.experimental.pallas import tpu as pltpu
from jax.experimental.pallas import tpu_sc as plsc

_B = 16
_D = 768
_RES = 512
_K = 64
_FG = 32
_INV_T = float(1.0 / 0.3)
_ROWS = 2048
_LANES = 128
_GROUPS = 256

_NC = 2
_NS = 16
_NW = _NC * _NS
_R = _B * _K * 16
_RPW = _R // _NW
_CH = 64
_NCH = _RPW // _CH


def _stats_kernel(w_ref, selp_ref, rows_ref):
    b = pl.program_id(0)
    big = jnp.int32(1 << 30)
    neg = jnp.float32(-jnp.inf)
    giota = (lax.broadcasted_iota(jnp.int32, (2, 128), 0) * 128
             + lax.broadcasted_iota(jnp.int32, (2, 128), 1))

    def init_g(g, carry):
        gval, gidx = carry
        blk = w_ref[0, pl.ds(g * 8, 8), :]
        flat = ((g * 8 + lax.broadcasted_iota(jnp.int32, (8, 128), 0)) * 128
                + lax.broadcasted_iota(jnp.int32, (8, 128), 1))
        m = jnp.max(blk)
        mi = jnp.min(jnp.where(blk == m, flat, big))
        gval = jnp.where(giota == g, m, gval)
        gidx = jnp.where(giota == g, mi, gidx)
        return gval, gidx

    gval, gidx = lax.fori_loop(
        0, _GROUPS, init_g,
        (jnp.full((2, 128), neg, jnp.float32),
         jnp.full((2, 128), big, jnp.int32)))

    maxw = jnp.max(gval)
    inv_t = jnp.float32(_INV_T)

    def denom_step(i, acc):
        slab = w_ref[0, pl.ds(i * 256, 256), :]
        return acc + jnp.sum(jnp.exp((slab - maxw) * inv_t))

    denom = lax.fori_loop(0, 8, denom_step, jnp.float32(0.0))

    kiota = lax.broadcasted_iota(jnp.int32, (1, 64), 1)

    def extract(k, carry):
        gval, gidx, topv, topidx = carry
        m = jnp.max(gval)
        cand = jnp.min(jnp.where(gval == m, gidx, big))
        g = cand // 1024
        row0 = g * 8
        blk = w_ref[0, pl.ds(row0, 8), :]
        flat = ((row0 + lax.broadcasted_iota(jnp.int32, (8, 128), 0)) * 128
                + lax.broadcasted_iota(jnp.int32, (8, 128), 1))
        blk = jnp.where(flat == cand, neg, blk)
        w_ref[0, pl.ds(row0, 8), :] = blk
        nm = jnp.max(blk)
        ni = jnp.min(jnp.where(blk == nm, flat, big))
        gval = jnp.where(giota == g, nm, gval)
        gidx = jnp.where(giota == g, ni, gidx)
        topv = jnp.where(kiota == k, m, topv)
        topidx = jnp.where(kiota == k, cand, topidx)
        return gval, gidx, topv, topidx

    _, _, topv, topidx = lax.fori_loop(
        0, _K, extract,
        (gval, gidx, jnp.zeros((1, 64), jnp.float32),
         jnp.zeros((1, 64), jnp.int32)))

    selp_ref[0] = jnp.exp((topv - maxw) * inv_t) / denom
    h = topidx // _RES
    wp = topidx - h * _RES
    fh = jnp.minimum(h // 16, _FG - 4)
    fw = jnp.minimum(wp // 16, _FG - 4)
    base = b * (_FG * _FG) + fh * _FG + fw
    for t in range(16):
        oh, ow = t // 4, t % 4
        rows_ref[0, pl.ds(t, 1), :] = base + oh * _FG + ow


def _sc_gather_kernel(table_hbm, idx_hbm, pb_hbm, out_hbm,
                      idx_v, pb_v, rows_v, sem):
    wid = lax.axis_index("s") * _NC + lax.axis_index("c")

    def chunk(i, carry):
        base = wid * _RPW + i * _CH
        pltpu.sync_copy(idx_hbm.at[pl.ds(base, _CH)], idx_v)
        pltpu.sync_copy(pb_hbm.at[pl.ds(base, _CH)], pb_v)
        pltpu.async_copy(table_hbm.at[idx_v], rows_v, sem).wait()
        for r in range(_CH):
            p = pb_v[r, :]

            def scale(j, c):
                rows_v[r, pl.ds(j * 64, 16)] = rows_v[r, pl.ds(j * 64, 16)] * p
                rows_v[r, pl.ds(j * 64 + 16, 16)] = (
                    rows_v[r, pl.ds(j * 64 + 16, 16)] * p)
                rows_v[r, pl.ds(j * 64 + 32, 16)] = (
                    rows_v[r, pl.ds(j * 64 + 32, 16)] * p)
                rows_v[r, pl.ds(j * 64 + 48, 16)] = (
                    rows_v[r, pl.ds(j * 64 + 48, 16)] * p)
                return c

            lax.fori_loop(0, _D // 64, scale, 0)
        pltpu.sync_copy(rows_v, out_hbm.at[pl.ds(base, _CH)])
        return carry

    lax.fori_loop(0, _NCH, chunk, 0)


def _sc_gather(table, idx, probs):
    return pl.kernel(
        _sc_gather_kernel,
        out_type=jax.ShapeDtypeStruct((_R, _D), jnp.float32),
        mesh=plsc.VectorSubcoreMesh(core_axis_name="c", subcore_axis_name="s"),
        scratch_types=[
            pltpu.VMEM((_CH,), jnp.int32),
            pltpu.VMEM((_CH, 16), jnp.float32),
            pltpu.VMEM((_CH, _D), jnp.float32),
            pltpu.SemaphoreType.DMA,
        ],
    )(table, idx, probs)


@jax.jit
def kernel(features, weights):
    w3 = weights.reshape(_B, _ROWS, _LANES)
    selp, rows16 = pl.pallas_call(
        _stats_kernel,
        grid=(_B,),
        in_specs=[pl.BlockSpec((1, _ROWS, _LANES), lambda b: (b, 0, 0))],
        out_specs=[pl.BlockSpec((1, 1, 64), lambda b: (b, 0, 0)),
                   pl.BlockSpec((1, 16, 64), lambda b: (b, 0, 0))],
        out_shape=[jax.ShapeDtypeStruct((_B, 1, 64), jnp.float32),
                   jax.ShapeDtypeStruct((_B, 16, 64), jnp.int32)],
    )(w3)

    idx = rows16.transpose(0, 2, 1).reshape(_R)
    probs = jnp.broadcast_to(selp.reshape(_B, _K, 1, 1),
                             (_B, _K, 16, 16)).reshape(_R, 16)
    table = features.reshape(_B * _FG * _FG, _D)
    out = _sc_gather(table, idx, probs)
    return out.reshape(_B, _K, 16, _D)


# speedup vs baseline: 3.8577x; 3.8577x over previous
import functools

import jax
import jax.numpy as jnp
from jax import lax
from jax.experimental import pallas as pl
from jax.experimental.pallas import tpu as pltpu
from jax.experimental.pallas import tpu_sc as plsc

_B = 16
_D = 768
_RES = 512
_K = 64
_FG = 32
_INV_T = float(1.0 / 0.3)
_ROWS = 2048
_LANES = 128
_GROUPS = 256

_NC = 2
_NS = 16
_NW = _NC * _NS
_R = _B * _K * 16
_RPW = _R // _NW
_CH = 64
_NCH = _RPW // _CH


def _stats_kernel(w_ref, selp_ref, rows_ref, m1_ref, mi_ref):
    big = jnp.int32(1 << 30)
    neg = jnp.float32(-jnp.inf)
    inv_t = jnp.float32(_INV_T)
    kiota = lax.broadcasted_iota(jnp.int32, (1, 64), 1)
    r8 = lax.broadcasted_iota(jnp.int32, (8, 128), 0)
    l8 = lax.broadcasted_iota(jnp.int32, (8, 128), 1)
    ciota = lax.broadcasted_iota(jnp.int32, (256, 128), 0)
    liota = lax.broadcasted_iota(jnp.int32, (256, 128), 1)
    g8 = lax.broadcasted_iota(jnp.int32, (8, 8, 128), 1)

    for b in range(_B):
        def init_w(gg, c):
            blk = w_ref[b, pl.ds(gg * 64, 64), :].reshape(8, 8, 128)
            nv = jnp.max(blk, axis=1)
            ni = jnp.min(jnp.where(blk == nv[:, None, :], g8, 8), axis=1)
            m1_ref[b, pl.ds(gg * 8, 8), :] = nv
            mi_ref[b, pl.ds(gg * 8, 8), :] = ni
            return c

        lax.fori_loop(0, 32, init_w, 0)

    maxws, denoms = [], []
    for b in range(_B):
        maxw = jnp.max(m1_ref[b])

        def dstep(i, acc, b=b, maxw=maxw):
            slab = w_ref[b, pl.ds(i * 256, 256), :]
            return acc + jnp.sum(jnp.exp((slab - maxw) * inv_t))

        denoms.append(lax.fori_loop(0, 8, dstep, jnp.float32(0.0)))
        maxws.append(maxw)

    def extract(k, carry):
        tvs, tis = carry
        ntvs, ntis = [], []
        for b in range(_B):
            m1 = m1_ref[b]
            mi = mi_ref[b]
            m = jnp.max(m1)
            cand = jnp.min(jnp.where(
                m1 == m, (ciota * 8 + mi) * 128 + liota, big))
            ci = cand // 1024
            blk = w_ref[b, pl.ds(ci * 8, 8), :]
            flat = (ci * 8 + r8) * 128 + l8
            blk = jnp.where(flat == cand, neg, blk)
            w_ref[b, pl.ds(ci * 8, 8), :] = blk
            nv = jnp.max(blk, axis=0, keepdims=True)
            ni = jnp.min(jnp.where(blk == nv, r8, 8), axis=0, keepdims=True)
            ca = pl.multiple_of((ci // 8) * 8, 8)
            sel = r8 == (ci - ca)
            m1_ref[b, pl.ds(ca, 8), :] = jnp.where(
                sel, nv, m1_ref[b, pl.ds(ca, 8), :])
            mi_ref[b, pl.ds(ca, 8), :] = jnp.where(
                sel, ni, mi_ref[b, pl.ds(ca, 8), :])
            ntvs.append(jnp.where(kiota == k, m, tvs[b]))
            ntis.append(jnp.where(kiota == k, cand, tis[b]))
        return tuple(ntvs), tuple(ntis)

    z = tuple(jnp.zeros((1, 64), jnp.float32) for _ in range(_B))
    zi = tuple(jnp.zeros((1, 64), jnp.int32) for _ in range(_B))
    tvs, tis = lax.fori_loop(0, _K, extract, (z, zi))

    for b in range(_B):
        selp_ref[b] = jnp.exp((tvs[b] - maxws[b]) * inv_t) / denoms[b]
        h = tis[b] // _RES
        wp = tis[b] - h * _RES
        fh = jnp.minimum(h // 16, _FG - 4)
        fw = jnp.minimum(wp // 16, _FG - 4)
        base = b * (_FG * _FG) + fh * _FG + fw
        for t in range(16):
            rows_ref[b, pl.ds(t, 1), :] = base + (t // 4) * _FG + (t % 4)


def _sc_gather_kernel(table_hbm, idx_hbm, pb_hbm, out_hbm,
                      idx_v, pb_v, rows_v, sem):
    wid = lax.axis_index("s") * _NC + lax.axis_index("c")

    def chunk(i, carry):
        base = wid * _RPW + i * _CH
        pltpu.sync_copy(idx_hbm.at[pl.ds(base, _CH)], idx_v)
        pltpu.sync_copy(pb_hbm.at[pl.ds(base, _CH)], pb_v)
        pltpu.async_copy(table_hbm.at[idx_v], rows_v, sem).wait()
        for r in range(_CH):
            p = pb_v[r, :]

            def scale(j, c):
                rows_v[r, pl.ds(j * 64, 16)] = rows_v[r, pl.ds(j * 64, 16)] * p
                rows_v[r, pl.ds(j * 64 + 16, 16)] = (
                    rows_v[r, pl.ds(j * 64 + 16, 16)] * p)
                rows_v[r, pl.ds(j * 64 + 32, 16)] = (
                    rows_v[r, pl.ds(j * 64 + 32, 16)] * p)
                rows_v[r, pl.ds(j * 64 + 48, 16)] = (
                    rows_v[r, pl.ds(j * 64 + 48, 16)] * p)
                return c

            lax.fori_loop(0, _D // 64, scale, 0)
        pltpu.sync_copy(rows_v, out_hbm.at[pl.ds(base, _CH)])
        return carry

    lax.fori_loop(0, _NCH, chunk, 0)


def _sc_gather(table, idx, probs):
    return pl.kernel(
        _sc_gather_kernel,
        out_type=jax.ShapeDtypeStruct((_R, _D), jnp.float32),
        mesh=plsc.VectorSubcoreMesh(core_axis_name="c", subcore_axis_name="s"),
        scratch_types=[
            pltpu.VMEM((_CH,), jnp.int32),
            pltpu.VMEM((_CH, 16), jnp.float32),
            pltpu.VMEM((_CH, _D), jnp.float32),
            pltpu.SemaphoreType.DMA,
        ],
    )(table, idx, probs)


@jax.jit
def kernel(features, weights):
    w3 = weights.reshape(_B, _ROWS, _LANES)
    selp, rows16 = pl.pallas_call(
        _stats_kernel,
        out_shape=[jax.ShapeDtypeStruct((_B, 1, 64), jnp.float32),
                   jax.ShapeDtypeStruct((_B, 16, 64), jnp.int32)],
        scratch_shapes=[pltpu.VMEM((_B, _GROUPS, _LANES), jnp.float32),
                        pltpu.VMEM((_B, _GROUPS, _LANES), jnp.int32)],
    )(w3)

    idx = rows16.transpose(0, 2, 1).reshape(_R)
    probs = jnp.broadcast_to(selp.reshape(_B, _K, 1, 1),
                             (_B, _K, 16, 16)).reshape(_R, 16)
    table = features.reshape(_B * _FG * _FG, _D)
    out = _sc_gather(table, idx, probs)
    return out.reshape(_B, _K, 16, _D)


# speedup vs baseline: 5.1259x; 1.3287x over previous
import functools

import jax
import jax.numpy as jnp
from jax import lax
from jax.experimental import pallas as pl
from jax.experimental.pallas import tpu as pltpu
from jax.experimental.pallas import tpu_sc as plsc

_B = 16
_D = 768
_RES = 512
_K = 64
_FG = 32
_INV_T = float(1.0 / 0.3)
_ROWS = 2048
_LANES = 128
_GROUPS = 256

_NC = 2
_NS = 16
_NW = _NC * _NS
_R = _B * _K * 16
_RPW = _R // _NW
_CH = 64
_NCH = _RPW // _CH


def _stats_kernel(w_ref, selp_ref, rows_ref, *scr):
    ws = scr[0:_B]
    m1s = scr[_B:2 * _B]
    mis = scr[2 * _B:3 * _B]
    big = jnp.int32(1 << 30)
    neg = jnp.float32(-jnp.inf)
    inv_t = jnp.float32(_INV_T)
    kiota = lax.broadcasted_iota(jnp.int32, (1, 64), 1)
    r8 = lax.broadcasted_iota(jnp.int32, (8, 128), 0)
    l8 = lax.broadcasted_iota(jnp.int32, (8, 128), 1)
    ciota = lax.broadcasted_iota(jnp.int32, (256, 128), 0)
    liota = lax.broadcasted_iota(jnp.int32, (256, 128), 1)
    g8 = lax.broadcasted_iota(jnp.int32, (8, 8, 128), 1)

    for b in range(_B):
        def init_w(gg, c, b=b):
            slab = w_ref[b, pl.ds(gg * 64, 64), :]
            ws[b][pl.ds(gg * 64, 64), :] = slab
            blk = slab.reshape(8, 8, 128)
            nv = jnp.max(blk, axis=1)
            ni = jnp.min(jnp.where(blk == nv[:, None, :], g8, 8), axis=1)
            m1s[b][pl.ds(gg * 8, 8), :] = nv
            mis[b][pl.ds(gg * 8, 8), :] = ni
            return c

        lax.fori_loop(0, 32, init_w, 0)

    maxws, denoms = [], []
    for b in range(_B):
        maxw = jnp.max(m1s[b][:, :])

        def dstep(i, acc, b=b, maxw=maxw):
            slab = w_ref[b, pl.ds(i * 256, 256), :]
            return acc + jnp.sum(jnp.exp((slab - maxw) * inv_t))

        denoms.append(lax.fori_loop(0, 8, dstep, jnp.float32(0.0)))
        maxws.append(maxw)

    def extract(k, carry):
        tvs, tis = carry
        ntvs, ntis = [], []
        for b in range(_B):
            m1 = m1s[b][:, :]
            mi = mis[b][:, :]
            m = jnp.max(m1)
            cand = jnp.min(jnp.where(
                m1 == m, (ciota * 8 + mi) * 128 + liota, big))
            ci = cand // 1024
            blk = ws[b][pl.ds(ci * 8, 8), :]
            flat = (ci * 8 + r8) * 128 + l8
            blk = jnp.where(flat == cand, neg, blk)
            ws[b][pl.ds(ci * 8, 8), :] = blk
            nv = jnp.max(blk, axis=0, keepdims=True)
            ni = jnp.min(jnp.where(blk == nv, r8, 8), axis=0, keepdims=True)
            ca = pl.multiple_of((ci // 8) * 8, 8)
            sel = r8 == (ci - ca)
            m1s[b][pl.ds(ca, 8), :] = jnp.where(
                sel, nv, m1s[b][pl.ds(ca, 8), :])
            mis[b][pl.ds(ca, 8), :] = jnp.where(
                sel, ni, mis[b][pl.ds(ca, 8), :])
            ntvs.append(jnp.where(kiota == k, m, tvs[b]))
            ntis.append(jnp.where(kiota == k, cand, tis[b]))
        return tuple(ntvs), tuple(ntis)

    z = tuple(jnp.zeros((1, 64), jnp.float32) for _ in range(_B))
    zi = tuple(jnp.zeros((1, 64), jnp.int32) for _ in range(_B))
    tvs, tis = lax.fori_loop(0, _K, extract, (z, zi))

    for b in range(_B):
        selp_ref[b] = jnp.exp((tvs[b] - maxws[b]) * inv_t) / denoms[b]
        h = tis[b] // _RES
        wp = tis[b] - h * _RES
        fh = jnp.minimum(h // 16, _FG - 4)
        fw = jnp.minimum(wp // 16, _FG - 4)
        base = b * (_FG * _FG) + fh * _FG + fw
        for t in range(16):
            rows_ref[b, pl.ds(t, 1), :] = base + (t // 4) * _FG + (t % 4)


def _sc_gather_kernel(table_hbm, idx_hbm, pb_hbm, out_hbm,
                      idx_v, pb_v, rows_v, sem):
    wid = lax.axis_index("s") * _NC + lax.axis_index("c")

    def chunk(i, carry):
        base = wid * _RPW + i * _CH
        pltpu.sync_copy(idx_hbm.at[pl.ds(base, _CH)], idx_v)
        pltpu.sync_copy(pb_hbm.at[pl.ds(base, _CH)], pb_v)
        pltpu.async_copy(table_hbm.at[idx_v], rows_v, sem).wait()
        for r in range(_CH):
            p = pb_v[r, :]

            def scale(j, c):
                rows_v[r, pl.ds(j * 64, 16)] = rows_v[r, pl.ds(j * 64, 16)] * p
                rows_v[r, pl.ds(j * 64 + 16, 16)] = (
                    rows_v[r, pl.ds(j * 64 + 16, 16)] * p)
                rows_v[r, pl.ds(j * 64 + 32, 16)] = (
                    rows_v[r, pl.ds(j * 64 + 32, 16)] * p)
                rows_v[r, pl.ds(j * 64 + 48, 16)] = (
                    rows_v[r, pl.ds(j * 64 + 48, 16)] * p)
                return c

            lax.fori_loop(0, _D // 64, scale, 0)
        pltpu.sync_copy(rows_v, out_hbm.at[pl.ds(base, _CH)])
        return carry

    lax.fori_loop(0, _NCH, chunk, 0)


def _sc_gather(table, idx, probs):
    return pl.kernel(
        _sc_gather_kernel,
        out_type=jax.ShapeDtypeStruct((_R, _D), jnp.float32),
        mesh=plsc.VectorSubcoreMesh(core_axis_name="c", subcore_axis_name="s"),
        scratch_types=[
            pltpu.VMEM((_CH,), jnp.int32),
            pltpu.VMEM((_CH, 16), jnp.float32),
            pltpu.VMEM((_CH, _D), jnp.float32),
            pltpu.SemaphoreType.DMA,
        ],
    )(table, idx, probs)


@jax.jit
def kernel(features, weights):
    w3 = weights.reshape(_B, _ROWS, _LANES)
    selp, rows16 = pl.pallas_call(
        _stats_kernel,
        out_shape=[jax.ShapeDtypeStruct((_B, 1, 64), jnp.float32),
                   jax.ShapeDtypeStruct((_B, 16, 64), jnp.int32)],
        scratch_shapes=(
            [pltpu.VMEM((_ROWS, _LANES), jnp.float32)] * _B
            + [pltpu.VMEM((_GROUPS, _LANES), jnp.float32)] * _B
            + [pltpu.VMEM((_GROUPS, _LANES), jnp.int32)] * _B),
    )(w3)

    idx = rows16.transpose(0, 2, 1).reshape(_R)
    probs = jnp.broadcast_to(selp.reshape(_B, _K, 1, 1),
                             (_B, _K, 16, 16)).reshape(_R, 16)
    table = features.reshape(_B * _FG * _FG, _D)
    out = _sc_gather(table, idx, probs)
    return out.reshape(_B, _K, 16, _D)
